# baseline (speedup 1.0000x reference)
import functools
import jax
import jax.numpy as jnp
from jax import lax
from jax.experimental import pallas as pl
from jax.experimental.pallas import tpu as pltpu

_N = 25000
_E = 400000
_H = 4
_TILE = 1000


def _zproj_body(x_src_ref, x_dst_ref, w_ref, al_ref, ar_ref, z_ref, elr_ref):
    x_s = x_src_ref[...]
    x_d = x_dst_ref[...]
    w = w_ref[...]
    al = al_ref[...]
    ar = ar_ref[...]
    z = jnp.dot(x_s, w, preferred_element_type=jnp.float32)
    z_ref[...] = z
    c = w.shape[0]
    h = al.shape[0]
    d = al.shape[1]
    wl = (w.reshape(c, h, d) * al[None, :, :]).sum(-1)
    wr = (w.reshape(c, h, d) * ar[None, :, :]).sum(-1)
    el = jnp.dot(x_s, wl, preferred_element_type=jnp.float32)
    er = jnp.dot(x_d, wr, preferred_element_type=jnp.float32)
    elr_ref[...] = jnp.concatenate([el, er], axis=1)


def _zproj(x_src, x_dst, w, al, ar):
    n = x_src.shape[0]
    c = x_src.shape[1]
    k = w.shape[1]
    grid = (n // _TILE,)
    return pl.pallas_call(
        _zproj_body,
        grid=grid,
        in_specs=[
            pl.BlockSpec((_TILE, c), lambda i: (i, 0)),
            pl.BlockSpec((_TILE, c), lambda i: (i, 0)),
            pl.BlockSpec((c, k), lambda i: (0, 0)),
            pl.BlockSpec((_H, al.shape[1]), lambda i: (0, 0)),
            pl.BlockSpec((_H, al.shape[1]), lambda i: (0, 0)),
        ],
        out_specs=[
            pl.BlockSpec((_TILE, k), lambda i: (i, 0)),
            pl.BlockSpec((_TILE, 2 * _H), lambda i: (i, 0)),
        ],
        out_shape=[
            jax.ShapeDtypeStruct((n, k), jnp.float32),
            jax.ShapeDtypeStruct((n, 2 * _H), jnp.float32),
        ],
    )(x_src, x_dst, w, al, ar)


def _gat_conv(x_src, x_dst, src, dst, w, al, ar, n_dst):
    d = al.shape[1]
    z, elr = _zproj(x_src, x_dst, w, al, ar)
    el_e = elr[src, :_H]
    er_e = elr[dst, _H:]
    s = el_e + er_e
    e = jnp.maximum(s, 0.2 * s)
    ee = jnp.exp(e)
    den = jax.ops.segment_sum(ee, dst, num_segments=n_dst)
    num = jax.ops.segment_sum(
        ee[:, :, None] * z[src].reshape(-1, _H, d), dst, num_segments=n_dst)
    den = jnp.where(den == 0.0, 1.0, den)
    out = jnp.maximum(num / den[:, :, None], 0.0)
    return out.reshape(n_dst, _H * d)


def kernel(h_p, h_d, edge_index_pd, edge_index_dp, W1_pd, al1_pd, ar1_pd,
           W1_dp, al1_dp, ar1_dp, W2_pd, al2_pd, ar2_pd, W2_dp, al2_dp,
           ar2_dp, W3_pd, al3_pd, ar3_pd, W3_dp, al3_dp, ar3_dp):
    ps, pdst = edge_index_pd[0], edge_index_pd[1]
    ds, ddst = edge_index_dp[0], edge_index_dp[1]
    h_d1 = _gat_conv(h_p, h_d, ps, pdst, W1_pd, al1_pd, ar1_pd, _N)
    h_p1 = _gat_conv(h_d, h_p, ds, ddst, W1_dp, al1_dp, ar1_dp, _N)
    h_d2 = _gat_conv(h_p1, h_d1, ps, pdst, W2_pd, al2_pd, ar2_pd, _N)
    h_p2 = _gat_conv(h_d1, h_p1, ds, ddst, W2_dp, al2_dp, ar2_dp, _N)
    h_d3 = _gat_conv(h_p2, h_d2, ps, pdst, W3_pd, al3_pd, ar3_pd, _N)
    h_p3 = _gat_conv(h_d2, h_p2, ds, ddst, W3_dp, al3_dp, ar3_dp, _N)
    return (h_p3.reshape(_N, _H, 128), h_d3.reshape(_N, _H, 128))


# speedup vs baseline: 48.1588x; 48.1588x over previous
import functools
import jax
import jax.numpy as jnp
from jax import lax
from jax.experimental import pallas as pl
from jax.experimental.pallas import tpu as pltpu
from jax.experimental.pallas import tpu_sc as plsc

_N = 25000
_NP = 25088
_E = 400000
_EP = _E + 256
_H = 4
_TILE = 1000
_R = 784
_G = 56


def _zproj_body(x_src_ref, x_dst_ref, w_ref, al_ref, ar_ref,
                z_ref, el_ref, er_ref):
    x_s = x_src_ref[...]
    x_d = x_dst_ref[...]
    w = w_ref[...]
    al = al_ref[...]
    ar = ar_ref[...]
    z_ref[...] = jnp.dot(x_s, w, preferred_element_type=jnp.float32)
    c = w.shape[0]
    h, d = al.shape
    wl = (w.reshape(c, h, d) * al[None, :, :]).sum(-1)
    wr = (w.reshape(c, h, d) * ar[None, :, :]).sum(-1)
    el = jnp.dot(x_s, wl, preferred_element_type=jnp.float32)
    er = jnp.dot(x_d, wr, preferred_element_type=jnp.float32)
    t = el.shape[0]
    el_ref[...] = jnp.concatenate(
        [el, jnp.zeros((t, 128 - h), jnp.float32)], axis=1)
    er_ref[...] = jnp.concatenate(
        [er, jnp.zeros((t, 16 - h), jnp.float32)], axis=1)


def _zproj(x_src, x_dst, w, al, ar):
    n, c = x_src.shape
    k = w.shape[1]
    return pl.pallas_call(
        _zproj_body,
        grid=(n // _TILE,),
        in_specs=[
            pl.BlockSpec((_TILE, c), lambda i: (i, 0)),
            pl.BlockSpec((_TILE, c), lambda i: (i, 0)),
            pl.BlockSpec((c, k), lambda i: (0, 0)),
            pl.BlockSpec((_H, k // _H), lambda i: (0, 0)),
            pl.BlockSpec((_H, k // _H), lambda i: (0, 0)),
        ],
        out_specs=[
            pl.BlockSpec((_TILE, k), lambda i: (i, 0)),
            pl.BlockSpec((_TILE, 128), lambda i: (i, 0)),
            pl.BlockSpec((_TILE, 16), lambda i: (i, 0)),
        ],
        out_shape=[
            jax.ShapeDtypeStruct((n, k), jnp.float32),
            jax.ShapeDtypeStruct((n, 128), jnp.float32),
            jax.ShapeDtypeStruct((n, 16), jnp.float32),
        ],
    )(x_src, x_dst, w, al, ar)


def _prep_edges(edge_index):
    src = edge_index[0].astype(jnp.int32)
    dst = edge_index[1].astype(jnp.int32)
    dst_s, src_s = lax.sort((dst, src), num_keys=1)
    nb = _NP // _G
    bnd = jnp.minimum(jnp.arange(nb + 1, dtype=jnp.int32) * _G, _N)
    offs = jnp.searchsorted(dst_s, bnd, side='left').astype(jnp.int32)
    offs = jnp.pad(offs, (0, 15))
    src_p = jnp.pad(src_s, (0, _EP - _E))
    dst_p = jnp.pad(dst_s, (0, _EP - _E))
    return src_p, dst_p, offs


def _sc_gat32(z, el_p, er_p, src_p, dst_p, offs):
    K = 128
    RS = 112
    mesh = plsc.VectorSubcoreMesh(core_axis_name="c", subcore_axis_name="s")

    @functools.partial(
        pl.kernel,
        out_type=jax.ShapeDtypeStruct((_NP, 128), jnp.float32),
        mesh=mesh,
        scratch_types=[
            pltpu.VMEM((RS, 128), jnp.float32),
            pltpu.VMEM((RS * 4 + 16,), jnp.float32),
            pltpu.VMEM((RS, 16), jnp.float32),
            pltpu.VMEM((K, 128), jnp.float32),
            pltpu.VMEM((K, 128), jnp.float32),
            pltpu.VMEM((K,), jnp.int32),
            pltpu.VMEM((464,), jnp.int32),
            pltpu.VMEM((K + 16,), jnp.int32),
            pltpu.SemaphoreType.DMA,
            pltpu.SemaphoreType.DMA,
        ],
    )
    def kern(z_h, el_h, er_h, src_h, dst_h, offs_h, out_h,
             acc, den, erl, zg, elg, src_v, offs_s, dst_sm, sem0, sem1):
        wid = lax.axis_index("s") * 2 + lax.axis_index("c")
        dbase0 = wid * _R
        pltpu.sync_copy(offs_h, offs_s)

        zeros = jnp.zeros((16,), jnp.float32)

        for ss in range(7):
            dbase = dbase0 + ss * RS
            pltpu.sync_copy(er_h.at[pl.ds(dbase, RS)], erl)
            e_lo = offs_s[pl.ds(14 * wid + 2 * ss, 16)][0]
            e_hi = offs_s[pl.ds(14 * wid + 2 * ss + 2, 16)][0]
            e_al = (e_lo // 8) * 8
            nchunk = (e_hi - e_al + K - 1) // K

            def zero_acc(r, _):
                for cc in range(8):
                    acc[r, pl.ds(16 * cc, 16)] = zeros
                return 0
            lax.fori_loop(0, RS, zero_acc, 0)

            def zero_den(i, _):
                den[pl.ds(16 * i, 16)] = zeros
                return 0
            lax.fori_loop(0, (RS * 4 + 16) // 16, zero_den, 0)

            def chunk_body(c, _):
                base = e_al + c * K
                pltpu.sync_copy(src_h.at[pl.ds(base, K)], src_v)
                pltpu.sync_copy(dst_h.at[pl.ds(base, K)],
                                dst_sm.at[pl.ds(0, K)])
                cp0 = pltpu.async_copy(z_h.at[src_v], zg, sem0)
                cp1 = pltpu.async_copy(el_h.at[src_v], elg, sem1)
                cp0.wait()
                cp1.wait()

                j0 = jnp.maximum(e_lo - base, 0)
                j1 = jnp.minimum(e_hi - base, K)

                def edge_body(j, _):
                    dloc = dst_sm[pl.ds(j, 16)][0] - dbase
                    s = elg[j, pl.ds(0, 16)] + erl[dloc, pl.ds(0, 16)]
                    e = jnp.maximum(s, 0.2 * s)
                    ee_v = jnp.exp(e)
                    fm4 = jnp.clip(4 - lax.iota(jnp.int32, 16), 0, 1
                                   ).astype(jnp.float32)
                    plsc.addupdate(den.at[pl.ds(dloc * 4, 16)], ee_v * fm4)
                    for cc in range(8):
                        a = ee_v[cc >> 1]
                        val = zg[j, pl.ds(16 * cc, 16)] * a
                        plsc.addupdate(acc.at[dloc, pl.ds(16 * cc, 16)], val)
                    return 0
                lax.fori_loop(j0, j1, edge_body, 0)
                return 0
            lax.fori_loop(0, nchunk, chunk_body, 0)

            def flush_body(r, _):
                dv = den[pl.ds(4 * r, 16)]
                for cc in range(8):
                    dh = jnp.maximum(dv[cc >> 1], 1e-30)
                    a = acc[r, pl.ds(16 * cc, 16)]
                    acc[r, pl.ds(16 * cc, 16)] = jnp.maximum(a, 0.0) / dh
                return 0
            lax.fori_loop(0, RS, flush_body, 0)
            pltpu.sync_copy(acc, out_h.at[pl.ds(dbase, RS)])

    return kern(z, el_p, er_p, src_p, dst_p, offs)


def _sc_gat128(z, el_p, er_p, src_p, dst_p, offs):
    K = 64
    RS = 112
    mesh = plsc.VectorSubcoreMesh(core_axis_name="c", subcore_axis_name="s")

    @functools.partial(
        pl.kernel,
        out_type=jax.ShapeDtypeStruct((_NP, 512), jnp.float32),
        mesh=mesh,
        scratch_types=[
            pltpu.VMEM((RS, 128), jnp.float32),
            pltpu.VMEM((RS, 128), jnp.float32),
            pltpu.VMEM((RS, 128), jnp.float32),
            pltpu.VMEM((RS, 128), jnp.float32),
            pltpu.VMEM((RS * 4 + 16,), jnp.float32),
            pltpu.VMEM((RS, 16), jnp.float32),
            pltpu.VMEM((K, 512), jnp.float32),
            pltpu.VMEM((K, 128), jnp.float32),
            pltpu.VMEM((K,), jnp.int32),
            pltpu.VMEM((464,), jnp.int32),
            pltpu.VMEM((K + 16,), jnp.int32),
            pltpu.SemaphoreType.DMA,
            pltpu.SemaphoreType.DMA,
        ],
    )
    def kern(z_h, el_h, er_h, src_h, dst_h, offs_h, out_h,
             acc0, acc1, acc2, acc3, den, erl, zg, elg, src_v, offs_s,
             dst_sm, sem0, sem1):
        accs = (acc0, acc1, acc2, acc3)
        wid = lax.axis_index("s") * 2 + lax.axis_index("c")
        dbase0 = wid * _R
        pltpu.sync_copy(offs_h, offs_s)

        zeros = jnp.zeros((16,), jnp.float32)

        for ss in range(7):
            dbase = dbase0 + ss * RS
            pltpu.sync_copy(er_h.at[pl.ds(dbase, RS)], erl)
            e_lo = offs_s[pl.ds(14 * wid + 2 * ss, 16)][0]
            e_hi = offs_s[pl.ds(14 * wid + 2 * ss + 2, 16)][0]
            e_al = (e_lo // 8) * 8
            nchunk = (e_hi - e_al + K - 1) // K

            def zero_acc(r, _):
                for hh in range(4):
                    for cc in range(8):
                        accs[hh][r, pl.ds(16 * cc, 16)] = zeros
                return 0
            lax.fori_loop(0, RS, zero_acc, 0)

            def zero_den(i, _):
                den[pl.ds(16 * i, 16)] = zeros
                return 0
            lax.fori_loop(0, (RS * 4 + 16) // 16, zero_den, 0)

            def chunk_body(c, _):
                base = e_al + c * K
                pltpu.sync_copy(src_h.at[pl.ds(base, K)], src_v)
                pltpu.sync_copy(dst_h.at[pl.ds(base, K)],
                                dst_sm.at[pl.ds(0, K)])
                cp0 = pltpu.async_copy(z_h.at[src_v], zg, sem0)
                cp1 = pltpu.async_copy(el_h.at[src_v], elg, sem1)
                cp0.wait()
                cp1.wait()

                j0 = jnp.maximum(e_lo - base, 0)
                j1 = jnp.minimum(e_hi - base, K)

                def edge_body(j, _):
                    dloc = dst_sm[pl.ds(j, 16)][0] - dbase
                    s = elg[j, pl.ds(0, 16)] + erl[dloc, pl.ds(0, 16)]
                    e = jnp.maximum(s, 0.2 * s)
                    ee_v = jnp.exp(e)
                    fm4 = jnp.clip(4 - lax.iota(jnp.int32, 16), 0, 1
                                   ).astype(jnp.float32)
                    plsc.addupdate(den.at[pl.ds(dloc * 4, 16)], ee_v * fm4)
                    for hh in range(4):
                        a = ee_v[hh]
                        for cc in range(8):
                            val = zg[j, pl.ds(128 * hh + 16 * cc, 16)] * a
                            plsc.addupdate(
                                accs[hh].at[dloc, pl.ds(16 * cc, 16)], val)
                    return 0
                lax.fori_loop(j0, j1, edge_body, 0)
                return 0
            lax.fori_loop(0, nchunk, chunk_body, 0)

            def flush_body(r, _):
                dv = den[pl.ds(4 * r, 16)]
                for hh in range(4):
                    dh = jnp.maximum(dv[hh], 1e-30)
                    for cc in range(8):
                        a = accs[hh][r, pl.ds(16 * cc, 16)]
                        accs[hh][r, pl.ds(16 * cc, 16)] = (
                            jnp.maximum(a, 0.0) / dh)
                return 0
            lax.fori_loop(0, RS, flush_body, 0)
            for hh in range(4):
                pltpu.sync_copy(
                    accs[hh],
                    out_h.at[pl.ds(dbase, RS), pl.ds(128 * hh, 128)])

    return kern(z, el_p, er_p, src_p, dst_p, offs)


def _pad_n(x):
    return jnp.pad(x, ((0, _NP - _N), (0, 0)))


def _conv(x_src, x_dst, w, al, ar, edges):
    src_p, dst_p, offs = edges
    z, el, er = _zproj(x_src, x_dst, w, al, ar)
    sc = _sc_gat32 if w.shape[1] == 128 else _sc_gat128
    out = sc(z, _pad_n(el), _pad_n(er), src_p, dst_p, offs)
    return out[:_N]


def kernel(h_p, h_d, edge_index_pd, edge_index_dp, W1_pd, al1_pd, ar1_pd,
           W1_dp, al1_dp, ar1_dp, W2_pd, al2_pd, ar2_pd, W2_dp, al2_dp,
           ar2_dp, W3_pd, al3_pd, ar3_pd, W3_dp, al3_dp, ar3_dp):
    e_pd = _prep_edges(edge_index_pd)
    e_dp = _prep_edges(edge_index_dp)
    h_d1 = _conv(h_p, h_d, W1_pd, al1_pd, ar1_pd, e_pd)
    h_p1 = _conv(h_d, h_p, W1_dp, al1_dp, ar1_dp, e_dp)
    h_d2 = _conv(h_p1, h_d1, W2_pd, al2_pd, ar2_pd, e_pd)
    h_p2 = _conv(h_d1, h_p1, W2_dp, al2_dp, ar2_dp, e_dp)
    h_d3 = _conv(h_p2, h_d2, W3_pd, al3_pd, ar3_pd, e_pd)
    h_p3 = _conv(h_d2, h_p2, W3_dp, al3_dp, ar3_dp, e_dp)
    return (h_p3.reshape(_N, _H, 128), h_d3.reshape(_N, _H, 128))


# speedup vs baseline: 54.9495x; 1.1410x over previous
import functools
import jax
import jax.numpy as jnp
from jax import lax
from jax.experimental import pallas as pl
from jax.experimental.pallas import tpu as pltpu
from jax.experimental.pallas import tpu_sc as plsc

_N = 25000
_NP = 25088
_E = 400000
_EP = _E + 1024
_H = 4
_TILE = 1000
_R = 784
_G = 56


def _zproj_body(x_src_ref, x_dst_ref, w_ref, al_ref, ar_ref,
                z_ref, el_ref, er_ref):
    x_s = x_src_ref[...]
    x_d = x_dst_ref[...]
    w = w_ref[...]
    al = al_ref[...]
    ar = ar_ref[...]
    z_ref[...] = jnp.dot(x_s, w, preferred_element_type=jnp.float32)
    c = w.shape[0]
    h, d = al.shape
    wl = (w.reshape(c, h, d) * al[None, :, :]).sum(-1)
    wr = (w.reshape(c, h, d) * ar[None, :, :]).sum(-1)
    el = jnp.dot(x_s, wl, preferred_element_type=jnp.float32)
    er = jnp.dot(x_d, wr, preferred_element_type=jnp.float32)
    t = el.shape[0]
    el_ref[...] = jnp.concatenate(
        [el, jnp.zeros((t, 128 - h), jnp.float32)], axis=1)
    er_ref[...] = jnp.concatenate(
        [er, jnp.zeros((t, 16 - h), jnp.float32)], axis=1)


def _zproj(x_src, x_dst, w, al, ar):
    n, c = x_src.shape
    k = w.shape[1]
    return pl.pallas_call(
        _zproj_body,
        grid=(n // _TILE,),
        in_specs=[
            pl.BlockSpec((_TILE, c), lambda i: (i, 0)),
            pl.BlockSpec((_TILE, c), lambda i: (i, 0)),
            pl.BlockSpec((c, k), lambda i: (0, 0)),
            pl.BlockSpec((_H, k // _H), lambda i: (0, 0)),
            pl.BlockSpec((_H, k // _H), lambda i: (0, 0)),
        ],
        out_specs=[
            pl.BlockSpec((_TILE, k), lambda i: (i, 0)),
            pl.BlockSpec((_TILE, 128), lambda i: (i, 0)),
            pl.BlockSpec((_TILE, 16), lambda i: (i, 0)),
        ],
        out_shape=[
            jax.ShapeDtypeStruct((n, k), jnp.float32),
            jax.ShapeDtypeStruct((n, 128), jnp.float32),
            jax.ShapeDtypeStruct((n, 16), jnp.float32),
        ],
    )(x_src, x_dst, w, al, ar)


def _prep_edges(edge_index):
    src = edge_index[0].astype(jnp.int32)
    dst = edge_index[1].astype(jnp.int32)
    dst_s, src_s = lax.sort((dst, src), num_keys=1)
    nb = _NP // _G
    bnd = jnp.minimum(jnp.arange(nb + 1, dtype=jnp.int32) * _G, _N)
    offs = jnp.searchsorted(dst_s, bnd, side='left').astype(jnp.int32)
    offs = jnp.pad(offs, (0, 15))
    src_p = jnp.pad(src_s, (0, _EP - _E))
    dst_p = jnp.pad(dst_s, (0, _EP - _E))
    return src_p, dst_p, offs


def _sc_gat32(z, el_p, er_p, src_p, dst_p, offs):
    K = 128
    RS = 112
    mesh = plsc.VectorSubcoreMesh(core_axis_name="c", subcore_axis_name="s")

    @functools.partial(
        pl.kernel,
        out_type=jax.ShapeDtypeStruct((_NP, 128), jnp.float32),
        mesh=mesh,
        scratch_types=[
            pltpu.VMEM((RS, 128), jnp.float32),
            pltpu.VMEM((RS * 4 + 16,), jnp.float32),
            pltpu.VMEM((RS, 16), jnp.float32),
            pltpu.VMEM((K, 128), jnp.float32),
            pltpu.VMEM((K, 128), jnp.float32),
            pltpu.VMEM((K, 128), jnp.float32),
            pltpu.VMEM((K, 128), jnp.float32),
            pltpu.VMEM((K,), jnp.int32),
            pltpu.VMEM((K,), jnp.int32),
            pltpu.VMEM((464,), jnp.int32),
            pltpu.VMEM((K + 16,), jnp.int32),
            pltpu.VMEM((K + 16,), jnp.int32),
            pltpu.SemaphoreType.DMA,
            pltpu.SemaphoreType.DMA,
            pltpu.SemaphoreType.DMA,
            pltpu.SemaphoreType.DMA,
        ],
    )
    def kern(z_h, el_h, er_h, src_h, dst_h, offs_h, out_h,
             acc, den, erl, zgA, zgB, elgA, elgB, srcA, srcB, offs_s,
             dstA, dstB, semz0, seme0, semz1, seme1):
        zgs, elgs, srcs, dsts = (zgA, zgB), (elgA, elgB), (srcA, srcB), (dstA, dstB)
        sems = ((semz0, seme0), (semz1, seme1))
        wid = lax.axis_index("s") * 2 + lax.axis_index("c")
        dbase0 = wid * _R
        pltpu.sync_copy(offs_h, offs_s)

        zeros = jnp.zeros((16,), jnp.float32)

        def subpass(ss, _):
            dbase = dbase0 + ss * RS
            pltpu.sync_copy(er_h.at[pl.ds(dbase, RS)], erl)
            e_lo = offs_s[pl.ds(14 * wid + 2 * ss, 16)][0]
            e_hi = offs_s[pl.ds(14 * wid + 2 * ss + 2, 16)][0]
            e_al = (e_lo // 8) * 8
            nchunk = (e_hi - e_al + K - 1) // K

            def zero_acc(r, _):
                for cc in range(8):
                    acc[r, pl.ds(16 * cc, 16)] = zeros
                return 0
            lax.fori_loop(0, RS, zero_acc, 0)

            def zero_den(i, _):
                den[pl.ds(16 * i, 16)] = zeros
                return 0
            lax.fori_loop(0, (RS * 4 + 16) // 16, zero_den, 0)

            def issue(c, b):
                base = e_al + c * K
                pltpu.sync_copy(src_h.at[pl.ds(base, K)], srcs[b])
                pltpu.sync_copy(dst_h.at[pl.ds(base, K)],
                                dsts[b].at[pl.ds(0, K)])
                pltpu.async_copy(z_h.at[srcs[b]], zgs[b], sems[b][0])
                pltpu.async_copy(el_h.at[srcs[b]], elgs[b], sems[b][1])

            def wait(b):
                pltpu.make_async_copy(
                    z_h.at[srcs[b]], zgs[b], sems[b][0]).wait()
                pltpu.make_async_copy(
                    el_h.at[srcs[b]], elgs[b], sems[b][1]).wait()

            def process(c, b):
                base = e_al + c * K
                j0 = jnp.maximum(e_lo - base, 0)
                j1 = jnp.minimum(e_hi - base, K)

                def edge_body(j, _):
                    dloc = dsts[b][pl.ds(j, 16)][0] - dbase
                    s = elgs[b][j, pl.ds(0, 16)] + erl[dloc, pl.ds(0, 16)]
                    e = jnp.maximum(s, 0.2 * s)
                    ee_v = jnp.exp(e)
                    fm4 = jnp.clip(4 - lax.iota(jnp.int32, 16), 0, 1
                                   ).astype(jnp.float32)
                    plsc.addupdate(den.at[pl.ds(dloc * 4, 16)], ee_v * fm4)
                    for cc in range(8):
                        a = ee_v[cc >> 1]
                        val = zgs[b][j, pl.ds(16 * cc, 16)] * a
                        plsc.addupdate(acc.at[dloc, pl.ds(16 * cc, 16)], val)
                    return 0
                lax.fori_loop(j0, j1, edge_body, 0)

            issue(0, 0)
            nhalf = (nchunk + 1) // 2

            def pair_body(p, _):
                issue(2 * p + 1, 1)
                wait(0)
                process(2 * p, 0)
                issue(2 * p + 2, 0)
                wait(1)
                process(2 * p + 1, 1)
                return 0
            lax.fori_loop(0, nhalf, pair_body, 0)
            wait(0)

            def flush_body(r, _):
                dv = den[pl.ds(4 * r, 16)]
                for cc in range(8):
                    dh = jnp.maximum(dv[cc >> 1], 1e-30)
                    a = acc[r, pl.ds(16 * cc, 16)]
                    acc[r, pl.ds(16 * cc, 16)] = jnp.maximum(a, 0.0) / dh
                return 0
            lax.fori_loop(0, RS, flush_body, 0)
            pltpu.sync_copy(acc, out_h.at[pl.ds(dbase, RS)])
            return 0
        lax.fori_loop(0, 7, subpass, 0)

    return kern(z, el_p, er_p, src_p, dst_p, offs)


def _sc_gat128(z, el_p, er_p, src_p, dst_p, offs):
    K = 64
    RS = 56
    mesh = plsc.VectorSubcoreMesh(core_axis_name="c", subcore_axis_name="s")

    @functools.partial(
        pl.kernel,
        out_type=jax.ShapeDtypeStruct((_NP, 512), jnp.float32),
        mesh=mesh,
        scratch_types=[
            pltpu.VMEM((RS, 128), jnp.float32),
            pltpu.VMEM((RS, 128), jnp.float32),
            pltpu.VMEM((RS, 128), jnp.float32),
            pltpu.VMEM((RS, 128), jnp.float32),
            pltpu.VMEM((RS * 4 + 16,), jnp.float32),
            pltpu.VMEM((RS, 16), jnp.float32),
            pltpu.VMEM((K, 512), jnp.float32),
            pltpu.VMEM((K, 512), jnp.float32),
            pltpu.VMEM((K, 128), jnp.float32),
            pltpu.VMEM((K, 128), jnp.float32),
            pltpu.VMEM((K,), jnp.int32),
            pltpu.VMEM((K,), jnp.int32),
            pltpu.VMEM((464,), jnp.int32),
            pltpu.VMEM((K + 16,), jnp.int32),
            pltpu.VMEM((K + 16,), jnp.int32),
            pltpu.SemaphoreType.DMA,
            pltpu.SemaphoreType.DMA,
            pltpu.SemaphoreType.DMA,
            pltpu.SemaphoreType.DMA,
        ],
    )
    def kern(z_h, el_h, er_h, src_h, dst_h, offs_h, out_h,
             acc0, acc1, acc2, acc3, den, erl, zgA, zgB, elgA, elgB,
             srcA, srcB, offs_s, dstA, dstB, semz0, seme0, semz1, seme1):
        accs = (acc0, acc1, acc2, acc3)
        zgs, elgs, srcs, dsts = (zgA, zgB), (elgA, elgB), (srcA, srcB), (dstA, dstB)
        sems = ((semz0, seme0), (semz1, seme1))
        wid = lax.axis_index("s") * 2 + lax.axis_index("c")
        dbase0 = wid * _R
        pltpu.sync_copy(offs_h, offs_s)

        zeros = jnp.zeros((16,), jnp.float32)

        def subpass(ss, _):
            dbase = dbase0 + ss * RS
            pltpu.sync_copy(er_h.at[pl.ds(dbase, RS)], erl)
            e_lo = offs_s[pl.ds(14 * wid + ss, 16)][0]
            e_hi = offs_s[pl.ds(14 * wid + ss + 1, 16)][0]
            e_al = (e_lo // 8) * 8
            nchunk = (e_hi - e_al + K - 1) // K

            def zero_acc(r, _):
                for hh in range(4):
                    for cc in range(8):
                        accs[hh][r, pl.ds(16 * cc, 16)] = zeros
                return 0
            lax.fori_loop(0, RS, zero_acc, 0)

            def zero_den(i, _):
                den[pl.ds(16 * i, 16)] = zeros
                return 0
            lax.fori_loop(0, (RS * 4 + 16) // 16, zero_den, 0)

            def issue(c, b):
                base = e_al + c * K
                pltpu.sync_copy(src_h.at[pl.ds(base, K)], srcs[b])
                pltpu.sync_copy(dst_h.at[pl.ds(base, K)],
                                dsts[b].at[pl.ds(0, K)])
                pltpu.async_copy(z_h.at[srcs[b]], zgs[b], sems[b][0])
                pltpu.async_copy(el_h.at[srcs[b]], elgs[b], sems[b][1])

            def wait(b):
                pltpu.make_async_copy(
                    z_h.at[srcs[b]], zgs[b], sems[b][0]).wait()
                pltpu.make_async_copy(
                    el_h.at[srcs[b]], elgs[b], sems[b][1]).wait()

            def process(c, b):
                base = e_al + c * K
                j0 = jnp.maximum(e_lo - base, 0)
                j1 = jnp.minimum(e_hi - base, K)

                def edge_body(j, _):
                    dloc = dsts[b][pl.ds(j, 16)][0] - dbase
                    s = elgs[b][j, pl.ds(0, 16)] + erl[dloc, pl.ds(0, 16)]
                    e = jnp.maximum(s, 0.2 * s)
                    ee_v = jnp.exp(e)
                    fm4 = jnp.clip(4 - lax.iota(jnp.int32, 16), 0, 1
                                   ).astype(jnp.float32)
                    plsc.addupdate(den.at[pl.ds(dloc * 4, 16)], ee_v * fm4)
                    for hh in range(4):
                        a = ee_v[hh]
                        for cc in range(8):
                            val = zgs[b][j, pl.ds(128 * hh + 16 * cc, 16)] * a
                            plsc.addupdate(
                                accs[hh].at[dloc, pl.ds(16 * cc, 16)], val)
                    return 0
                lax.fori_loop(j0, j1, edge_body, 0)

            issue(0, 0)
            nhalf = (nchunk + 1) // 2

            def pair_body(p, _):
                issue(2 * p + 1, 1)
                wait(0)
                process(2 * p, 0)
                issue(2 * p + 2, 0)
                wait(1)
                process(2 * p + 1, 1)
                return 0
            lax.fori_loop(0, nhalf, pair_body, 0)
            wait(0)

            def flush_body(r, _):
                dv = den[pl.ds(4 * r, 16)]
                for hh in range(4):
                    dh = jnp.maximum(dv[hh], 1e-30)
                    for cc in range(8):
                        a = accs[hh][r, pl.ds(16 * cc, 16)]
                        accs[hh][r, pl.ds(16 * cc, 16)] = (
                            jnp.maximum(a, 0.0) / dh)
                return 0
            lax.fori_loop(0, RS, flush_body, 0)
            for hh in range(4):
                pltpu.sync_copy(
                    accs[hh],
                    out_h.at[pl.ds(dbase, RS), pl.ds(128 * hh, 128)])
            return 0
        lax.fori_loop(0, 14, subpass, 0)

    return kern(z, el_p, er_p, src_p, dst_p, offs)


def _pad_n(x):
    return jnp.pad(x, ((0, _NP - _N), (0, 0)))


def _conv(x_src, x_dst, w, al, ar, edges):
    src_p, dst_p, offs = edges
    z, el, er = _zproj(x_src, x_dst, w, al, ar)
    sc = _sc_gat32 if w.shape[1] == 128 else _sc_gat128
    out = sc(z, _pad_n(el), _pad_n(er), src_p, dst_p, offs)
    return out[:_N]


def kernel(h_p, h_d, edge_index_pd, edge_index_dp, W1_pd, al1_pd, ar1_pd,
           W1_dp, al1_dp, ar1_dp, W2_pd, al2_pd, ar2_pd, W2_dp, al2_dp,
           ar2_dp, W3_pd, al3_pd, ar3_pd, W3_dp, al3_dp, ar3_dp):
    e_pd = _prep_edges(edge_index_pd)
    e_dp = _prep_edges(edge_index_dp)
    h_d1 = _conv(h_p, h_d, W1_pd, al1_pd, ar1_pd, e_pd)
    h_p1 = _conv(h_d, h_p, W1_dp, al1_dp, ar1_dp, e_dp)
    h_d2 = _conv(h_p1, h_d1, W2_pd, al2_pd, ar2_pd, e_pd)
    h_p2 = _conv(h_d1, h_p1, W2_dp, al2_dp, ar2_dp, e_dp)
    h_d3 = _conv(h_p2, h_d2, W3_pd, al3_pd, ar3_pd, e_pd)
    h_p3 = _conv(h_d2, h_p2, W3_dp, al3_dp, ar3_dp, e_dp)
    return (h_p3.reshape(_N, _H, 128), h_d3.reshape(_N, _H, 128))


# speedup vs baseline: 122.6494x; 2.2320x over previous
import functools
import jax
import jax.numpy as jnp
from jax import lax
from jax.experimental import pallas as pl
from jax.experimental.pallas import tpu as pltpu
from jax.experimental.pallas import tpu_sc as plsc

_N = 25000
_NP = 25088
_E = 400000
_EP = _E + 1024
_H = 4
_TILE = 1000
_R = 784
_G = 56


def _zproj_body(x_src_ref, x_dst_ref, w_ref, al_ref, ar_ref,
                z_ref, el_ref, er_ref):
    x_s = x_src_ref[...]
    x_d = x_dst_ref[...]
    w = w_ref[...]
    al = al_ref[...]
    ar = ar_ref[...]
    z_ref[...] = jnp.dot(x_s, w, preferred_element_type=jnp.float32)
    c = w.shape[0]
    h, d = al.shape
    wl = (w.reshape(c, h, d) * al[None, :, :]).sum(-1)
    wr = (w.reshape(c, h, d) * ar[None, :, :]).sum(-1)
    el = jnp.dot(x_s, wl, preferred_element_type=jnp.float32)
    er = jnp.dot(x_d, wr, preferred_element_type=jnp.float32)
    t = el.shape[0]
    el_ref[...] = jnp.concatenate(
        [el, jnp.zeros((t, 128 - h), jnp.float32)], axis=1)
    er_ref[...] = jnp.concatenate(
        [er, jnp.zeros((t, 16 - h), jnp.float32)], axis=1)


def _zproj(x_src, x_dst, w, al, ar):
    n, c = x_src.shape
    k = w.shape[1]
    return pl.pallas_call(
        _zproj_body,
        grid=(n // _TILE,),
        in_specs=[
            pl.BlockSpec((_TILE, c), lambda i: (i, 0)),
            pl.BlockSpec((_TILE, c), lambda i: (i, 0)),
            pl.BlockSpec((c, k), lambda i: (0, 0)),
            pl.BlockSpec((_H, k // _H), lambda i: (0, 0)),
            pl.BlockSpec((_H, k // _H), lambda i: (0, 0)),
        ],
        out_specs=[
            pl.BlockSpec((_TILE, k), lambda i: (i, 0)),
            pl.BlockSpec((_TILE, 128), lambda i: (i, 0)),
            pl.BlockSpec((_TILE, 16), lambda i: (i, 0)),
        ],
        out_shape=[
            jax.ShapeDtypeStruct((n, k), jnp.float32),
            jax.ShapeDtypeStruct((n, 128), jnp.float32),
            jax.ShapeDtypeStruct((n, 16), jnp.float32),
        ],
    )(x_src, x_dst, w, al, ar)


def _prep_edges(edge_index):
    src = edge_index[0].astype(jnp.int32)
    dst = edge_index[1].astype(jnp.int32)
    dst_s, src_s = lax.sort((dst, src), num_keys=1)
    nb = _NP // _G
    bnd = jnp.minimum(jnp.arange(nb + 1, dtype=jnp.int32) * _G, _N)
    offs = jnp.searchsorted(dst_s, bnd, side='left').astype(jnp.int32)
    offs = jnp.pad(offs, (0, 15))
    src_p = jnp.pad(src_s, (0, _EP - _E))
    dst_p = jnp.pad(dst_s, (0, _EP - _E))
    return src_p, dst_p, offs


def _sc_gat32(z, el_p, er_p, src_p, dst_p, offs):
    K = 128
    RS = 112
    mesh = plsc.VectorSubcoreMesh(core_axis_name="c", subcore_axis_name="s")

    @functools.partial(
        pl.kernel,
        out_type=jax.ShapeDtypeStruct((_NP, 128), jnp.float32),
        mesh=mesh,
        scratch_types=[
            pltpu.VMEM((RS, 128), jnp.float32),
            pltpu.VMEM((RS * 4 + 16,), jnp.float32),
            pltpu.VMEM((RS, 16), jnp.float32),
            pltpu.VMEM((K, 128), jnp.float32),
            pltpu.VMEM((K, 128), jnp.float32),
            pltpu.VMEM((K, 128), jnp.float32),
            pltpu.VMEM((K, 128), jnp.float32),
            pltpu.VMEM((K,), jnp.int32),
            pltpu.VMEM((K,), jnp.int32),
            pltpu.VMEM((464,), jnp.int32),
            pltpu.VMEM((K + 16,), jnp.int32),
            pltpu.VMEM((K + 16,), jnp.int32),
            pltpu.SemaphoreType.DMA,
            pltpu.SemaphoreType.DMA,
            pltpu.SemaphoreType.DMA,
            pltpu.SemaphoreType.DMA,
        ],
    )
    def kern(z_h, el_h, er_h, src_h, dst_h, offs_h, out_h,
             acc, den, erl, zgA, zgB, elgA, elgB, srcA, srcB, offs_s,
             dstA, dstB, semz0, seme0, semz1, seme1):
        zgs, elgs, srcs, dsts = (zgA, zgB), (elgA, elgB), (srcA, srcB), (dstA, dstB)
        sems = ((semz0, seme0), (semz1, seme1))
        wid = lax.axis_index("s") * 2 + lax.axis_index("c")
        dbase0 = wid * _R
        pltpu.sync_copy(offs_h, offs_s)

        zeros = jnp.zeros((16,), jnp.float32)

        def subpass(ss, _):
            dbase = dbase0 + ss * RS
            pltpu.sync_copy(er_h.at[pl.ds(dbase, RS)], erl)
            e_lo = offs_s[pl.ds(14 * wid + 2 * ss, 16)][0]
            e_hi = offs_s[pl.ds(14 * wid + 2 * ss + 2, 16)][0]
            e_al = (e_lo // 8) * 8
            nchunk = (e_hi - e_al + K - 1) // K

            def zero_acc(r, _):
                for cc in range(8):
                    acc[r, pl.ds(16 * cc, 16)] = zeros
                return 0
            lax.fori_loop(0, RS, zero_acc, 0)

            def zero_den(i, _):
                den[pl.ds(16 * i, 16)] = zeros
                return 0
            lax.fori_loop(0, (RS * 4 + 16) // 16, zero_den, 0)

            def issue(c, b):
                base = e_al + c * K
                pltpu.sync_copy(src_h.at[pl.ds(base, K)], srcs[b])
                pltpu.sync_copy(dst_h.at[pl.ds(base, K)],
                                dsts[b].at[pl.ds(0, K)])
                pltpu.async_copy(z_h.at[srcs[b]], zgs[b], sems[b][0])
                pltpu.async_copy(el_h.at[srcs[b]], elgs[b], sems[b][1])

            def wait(b):
                pltpu.make_async_copy(
                    z_h.at[srcs[b]], zgs[b], sems[b][0]).wait()
                pltpu.make_async_copy(
                    el_h.at[srcs[b]], elgs[b], sems[b][1]).wait()

            def process(c, b):
                base = e_al + c * K
                j0 = jnp.maximum(e_lo - base, 0)
                j1 = jnp.minimum(e_hi - base, K)

                @plsc.parallel_loop(j0, j1, unroll=4)
                def edge_body(j):
                    dloc = dsts[b][pl.ds(j, 16)][0] - dbase
                    s = elgs[b][j, pl.ds(0, 16)] + erl[dloc, pl.ds(0, 16)]
                    e = jnp.maximum(s, 0.2 * s)
                    ee_v = jnp.exp(e)
                    fm4 = jnp.clip(4 - lax.iota(jnp.int32, 16), 0, 1
                                   ).astype(jnp.float32)
                    plsc.addupdate(den.at[pl.ds(dloc * 4, 16)], ee_v * fm4)
                    for cc in range(8):
                        a = ee_v[cc >> 1]
                        val = zgs[b][j, pl.ds(16 * cc, 16)] * a
                        plsc.addupdate(acc.at[dloc, pl.ds(16 * cc, 16)], val)

            issue(0, 0)
            nhalf = (nchunk + 1) // 2

            def pair_body(p, _):
                issue(2 * p + 1, 1)
                wait(0)
                process(2 * p, 0)
                issue(2 * p + 2, 0)
                wait(1)
                process(2 * p + 1, 1)
                return 0
            lax.fori_loop(0, nhalf, pair_body, 0)
            wait(0)

            def flush_body(r, _):
                dv = den[pl.ds(4 * r, 16)]
                for cc in range(8):
                    dh = jnp.maximum(dv[cc >> 1], 1e-30)
                    a = acc[r, pl.ds(16 * cc, 16)]
                    acc[r, pl.ds(16 * cc, 16)] = jnp.maximum(a, 0.0) / dh
                return 0
            lax.fori_loop(0, RS, flush_body, 0)
            pltpu.sync_copy(acc, out_h.at[pl.ds(dbase, RS)])
            return 0
        lax.fori_loop(0, 7, subpass, 0)

    return kern(z, el_p, er_p, src_p, dst_p, offs)


def _sc_gat128(z, el_p, er_p, src_p, dst_p, offs):
    K = 64
    RS = 56
    mesh = plsc.VectorSubcoreMesh(core_axis_name="c", subcore_axis_name="s")

    @functools.partial(
        pl.kernel,
        out_type=jax.ShapeDtypeStruct((_NP, 512), jnp.float32),
        mesh=mesh,
        scratch_types=[
            pltpu.VMEM((RS, 128), jnp.float32),
            pltpu.VMEM((RS, 128), jnp.float32),
            pltpu.VMEM((RS, 128), jnp.float32),
            pltpu.VMEM((RS, 128), jnp.float32),
            pltpu.VMEM((RS * 4 + 16,), jnp.float32),
            pltpu.VMEM((RS, 16), jnp.float32),
            pltpu.VMEM((K, 512), jnp.float32),
            pltpu.VMEM((K, 512), jnp.float32),
            pltpu.VMEM((K, 128), jnp.float32),
            pltpu.VMEM((K, 128), jnp.float32),
            pltpu.VMEM((K,), jnp.int32),
            pltpu.VMEM((K,), jnp.int32),
            pltpu.VMEM((464,), jnp.int32),
            pltpu.VMEM((K + 16,), jnp.int32),
            pltpu.VMEM((K + 16,), jnp.int32),
            pltpu.SemaphoreType.DMA,
            pltpu.SemaphoreType.DMA,
            pltpu.SemaphoreType.DMA,
            pltpu.SemaphoreType.DMA,
        ],
    )
    def kern(z_h, el_h, er_h, src_h, dst_h, offs_h, out_h,
             acc0, acc1, acc2, acc3, den, erl, zgA, zgB, elgA, elgB,
             srcA, srcB, offs_s, dstA, dstB, semz0, seme0, semz1, seme1):
        accs = (acc0, acc1, acc2, acc3)
        zgs, elgs, srcs, dsts = (zgA, zgB), (elgA, elgB), (srcA, srcB), (dstA, dstB)
        sems = ((semz0, seme0), (semz1, seme1))
        wid = lax.axis_index("s") * 2 + lax.axis_index("c")
        dbase0 = wid * _R
        pltpu.sync_copy(offs_h, offs_s)

        zeros = jnp.zeros((16,), jnp.float32)

        def subpass(ss, _):
            dbase = dbase0 + ss * RS
            pltpu.sync_copy(er_h.at[pl.ds(dbase, RS)], erl)
            e_lo = offs_s[pl.ds(14 * wid + ss, 16)][0]
            e_hi = offs_s[pl.ds(14 * wid + ss + 1, 16)][0]
            e_al = (e_lo // 8) * 8
            nchunk = (e_hi - e_al + K - 1) // K

            def zero_acc(r, _):
                for hh in range(4):
                    for cc in range(8):
                        accs[hh][r, pl.ds(16 * cc, 16)] = zeros
                return 0
            lax.fori_loop(0, RS, zero_acc, 0)

            def zero_den(i, _):
                den[pl.ds(16 * i, 16)] = zeros
                return 0
            lax.fori_loop(0, (RS * 4 + 16) // 16, zero_den, 0)

            def issue(c, b):
                base = e_al + c * K
                pltpu.sync_copy(src_h.at[pl.ds(base, K)], srcs[b])
                pltpu.sync_copy(dst_h.at[pl.ds(base, K)],
                                dsts[b].at[pl.ds(0, K)])
                pltpu.async_copy(z_h.at[srcs[b]], zgs[b], sems[b][0])
                pltpu.async_copy(el_h.at[srcs[b]], elgs[b], sems[b][1])

            def wait(b):
                pltpu.make_async_copy(
                    z_h.at[srcs[b]], zgs[b], sems[b][0]).wait()
                pltpu.make_async_copy(
                    el_h.at[srcs[b]], elgs[b], sems[b][1]).wait()

            def process(c, b):
                base = e_al + c * K
                j0 = jnp.maximum(e_lo - base, 0)
                j1 = jnp.minimum(e_hi - base, K)

                @plsc.parallel_loop(j0, j1, unroll=2)
                def edge_body(j):
                    dloc = dsts[b][pl.ds(j, 16)][0] - dbase
                    s = elgs[b][j, pl.ds(0, 16)] + erl[dloc, pl.ds(0, 16)]
                    e = jnp.maximum(s, 0.2 * s)
                    ee_v = jnp.exp(e)
                    fm4 = jnp.clip(4 - lax.iota(jnp.int32, 16), 0, 1
                                   ).astype(jnp.float32)
                    plsc.addupdate(den.at[pl.ds(dloc * 4, 16)], ee_v * fm4)
                    for hh in range(4):
                        a = ee_v[hh]
                        for cc in range(8):
                            val = zgs[b][j, pl.ds(128 * hh + 16 * cc, 16)] * a
                            plsc.addupdate(
                                accs[hh].at[dloc, pl.ds(16 * cc, 16)], val)

            issue(0, 0)
            nhalf = (nchunk + 1) // 2

            def pair_body(p, _):
                issue(2 * p + 1, 1)
                wait(0)
                process(2 * p, 0)
                issue(2 * p + 2, 0)
                wait(1)
                process(2 * p + 1, 1)
                return 0
            lax.fori_loop(0, nhalf, pair_body, 0)
            wait(0)

            def flush_body(r, _):
                dv = den[pl.ds(4 * r, 16)]
                for hh in range(4):
                    dh = jnp.maximum(dv[hh], 1e-30)
                    for cc in range(8):
                        a = accs[hh][r, pl.ds(16 * cc, 16)]
                        accs[hh][r, pl.ds(16 * cc, 16)] = (
                            jnp.maximum(a, 0.0) / dh)
                return 0
            lax.fori_loop(0, RS, flush_body, 0)
            for hh in range(4):
                pltpu.sync_copy(
                    accs[hh],
                    out_h.at[pl.ds(dbase, RS), pl.ds(128 * hh, 128)])
            return 0
        lax.fori_loop(0, 14, subpass, 0)

    return kern(z, el_p, er_p, src_p, dst_p, offs)


def _pad_n(x):
    return jnp.pad(x, ((0, _NP - _N), (0, 0)))


def _conv(x_src, x_dst, w, al, ar, edges):
    src_p, dst_p, offs = edges
    z, el, er = _zproj(x_src, x_dst, w, al, ar)
    sc = _sc_gat32 if w.shape[1] == 128 else _sc_gat128
    out = sc(z, _pad_n(el), _pad_n(er), src_p, dst_p, offs)
    return out[:_N]


def kernel(h_p, h_d, edge_index_pd, edge_index_dp, W1_pd, al1_pd, ar1_pd,
           W1_dp, al1_dp, ar1_dp, W2_pd, al2_pd, ar2_pd, W2_dp, al2_dp,
           ar2_dp, W3_pd, al3_pd, ar3_pd, W3_dp, al3_dp, ar3_dp):
    e_pd = _prep_edges(edge_index_pd)
    e_dp = _prep_edges(edge_index_dp)
    h_d1 = _conv(h_p, h_d, W1_pd, al1_pd, ar1_pd, e_pd)
    h_p1 = _conv(h_d, h_p, W1_dp, al1_dp, ar1_dp, e_dp)
    h_d2 = _conv(h_p1, h_d1, W2_pd, al2_pd, ar2_pd, e_pd)
    h_p2 = _conv(h_d1, h_p1, W2_dp, al2_dp, ar2_dp, e_dp)
    h_d3 = _conv(h_p2, h_d2, W3_pd, al3_pd, ar3_pd, e_pd)
    h_p3 = _conv(h_d2, h_p2, W3_dp, al3_dp, ar3_dp, e_dp)
    return (h_p3.reshape(_N, _H, 128), h_d3.reshape(_N, _H, 128))
